# baseline (speedup 1.0000x reference)
import jax
import jax.numpy as jnp
from jax import lax
from jax.experimental import pallas as pl
from jax.experimental.pallas import tpu as pltpu
from jax.experimental.pallas import tpu_sc as plsc

_NC, _NS = 2, 16
_NW = _NC * _NS
_L = 16
_CB = 16


def _sc_lookup(table, x, b, s, v, d):
    bpw = b // _NW
    ipw = bpw * s
    nchunk = bpw // _CB
    mesh = plsc.VectorSubcoreMesh(
        core_axis_name="core", subcore_axis_name="subcore"
    )

    @pl.kernel(
        out_type=jax.ShapeDtypeStruct((b, s, d), jnp.float32),
        mesh=mesh,
        compiler_params=pltpu.CompilerParams(
            use_tc_tiling_on_sc=False, needs_layout_passes=False
        ),
        scratch_types=[
            pltpu.VMEM((v * d,), jnp.float32),
            pltpu.VMEM((bpw, s), jnp.int32),
            pltpu.VMEM((_CB, s, d), jnp.float32),
            pltpu.VMEM((_CB, s, d), jnp.float32),
            pltpu.SemaphoreType.DMA,
            pltpu.SemaphoreType.DMA,
        ],
    )
    def k(t_hbm, i_hbm, o_hbm, tab_v, idx_v, rows0, rows1, sem0, sem1):
        wid = lax.axis_index("subcore") * _NC + lax.axis_index("core")
        base_b = wid * bpw
        pltpu.sync_copy(t_hbm, tab_v)
        pltpu.sync_copy(i_hbm.at[pl.ds(base_b, bpw)], idx_v)

        rows = (rows0, rows1)
        sems = (sem0, sem1)

        def do_chunk(kc, buf):
            rb, sb = rows[buf], sems[buf]

            @pl.when(kc >= 2)
            def _():
                pltpu.make_async_copy(
                    rb, o_hbm.at[pl.ds(0, _CB)], sb
                ).wait()

            @plsc.parallel_loop(0, _CB, unroll=2)
            def _(g):
                ro = kc * _CB + g
                for g3 in range(s // _L):
                    idxv = idx_v[ro, pl.ds(g3 * _L, _L)] * d
                    for jj in range(_L):
                        a = idxv[jj]
                        sc = g3 * _L + jj
                        rb[g, sc, pl.ds(0, _L)] = tab_v[pl.ds(a, _L)]
                        rb[g, sc, pl.ds(_L, _L)] = tab_v[pl.ds(a + _L, _L)]
                rem = s % _L
                if rem:
                    idxv = idx_v[ro, pl.ds(s - _L, _L)] * d
                    for jj in range(_L - rem, _L):
                        a = idxv[jj]
                        sc = s - _L + jj
                        rb[g, sc, pl.ds(0, _L)] = tab_v[pl.ds(a, _L)]
                        rb[g, sc, pl.ds(_L, _L)] = tab_v[pl.ds(a + _L, _L)]

            pltpu.async_copy(
                rb, o_hbm.at[pl.ds(base_b + kc * _CB, _CB)], sb
            )

        @pl.loop(0, nchunk, step=2)
        def _(kc):
            do_chunk(kc, 0)
            do_chunk(kc + 1, 1)

        pltpu.make_async_copy(rows0, o_hbm.at[pl.ds(0, _CB)], sem0).wait()
        pltpu.make_async_copy(rows1, o_hbm.at[pl.ds(0, _CB)], sem1).wait()

    return k(table, x)


def kernel(x, table):
    b, s = x.shape
    v, d = table.shape
    return _sc_lookup(table.reshape(v * d), x, b, s, v, d)


# speedup vs baseline: 1.1270x; 1.1270x over previous
import jax
import jax.numpy as jnp
from jax import lax
from jax.experimental import pallas as pl
from jax.experimental.pallas import tpu as pltpu
from jax.experimental.pallas import tpu_sc as plsc

_NC, _NS = 2, 16
_NW = _NC * _NS
_L = 16
_CB = 4


def _sc_lookup(table, x, b, s, v, d):
    bpw = b // _NW
    ipw = bpw * s
    nchunk = bpw // _CB
    mesh = plsc.VectorSubcoreMesh(
        core_axis_name="core", subcore_axis_name="subcore"
    )

    @pl.kernel(
        out_type=jax.ShapeDtypeStruct((b, s, d), jnp.float32),
        mesh=mesh,
        compiler_params=pltpu.CompilerParams(
            use_tc_tiling_on_sc=True, needs_layout_passes=True
        ),
        scratch_types=[
            pltpu.VMEM((v * d,), jnp.float32),
            pltpu.VMEM((bpw, s), jnp.int32),
            pltpu.VMEM((_CB, s, d), jnp.float32),
            pltpu.VMEM((_CB, s, d), jnp.float32),
            pltpu.SemaphoreType.DMA,
            pltpu.SemaphoreType.DMA,
        ],
    )
    def k(t_hbm, i_hbm, o_hbm, tab_v, idx_v, rows0, rows1, sem0, sem1):
        wid = lax.axis_index("subcore") * _NC + lax.axis_index("core")
        base_b = wid * bpw
        pltpu.sync_copy(t_hbm, tab_v)
        pltpu.sync_copy(i_hbm.at[pl.ds(base_b, bpw)], idx_v)

        rows = (rows0, rows1)
        sems = (sem0, sem1)

        def do_chunk(kc, buf):
            rb, sb = rows[buf], sems[buf]

            @pl.when(kc >= 2)
            def _():
                pltpu.make_async_copy(
                    rb, o_hbm.at[pl.ds(0, _CB)], sb
                ).wait()

            @plsc.parallel_loop(0, _CB, unroll=2)
            def _(g):
                ro = kc * _CB + g
                for g3 in range(s // _L):
                    idxv = idx_v[ro, pl.ds(g3 * _L, _L)] * d
                    for jj in range(_L):
                        a = idxv[jj]
                        sc = g3 * _L + jj
                        rb[g, sc, pl.ds(0, _L)] = tab_v[pl.ds(a, _L)]
                        rb[g, sc, pl.ds(_L, _L)] = tab_v[pl.ds(a + _L, _L)]
                rem = s % _L
                if rem:
                    idxv = idx_v[ro, pl.ds(s - _L, _L)] * d
                    for jj in range(_L - rem, _L):
                        a = idxv[jj]
                        sc = s - _L + jj
                        rb[g, sc, pl.ds(0, _L)] = tab_v[pl.ds(a, _L)]
                        rb[g, sc, pl.ds(_L, _L)] = tab_v[pl.ds(a + _L, _L)]

            pltpu.async_copy(
                rb, o_hbm.at[pl.ds(base_b + kc * _CB, _CB)], sb
            )

        @pl.loop(0, nchunk, step=2)
        def _(kc):
            do_chunk(kc, 0)
            do_chunk(kc + 1, 1)

        pltpu.make_async_copy(rows0, o_hbm.at[pl.ds(0, _CB)], sem0).wait()
        pltpu.make_async_copy(rows1, o_hbm.at[pl.ds(0, _CB)], sem1).wait()

    return k(table, x)


def kernel(x, table):
    b, s = x.shape
    v, d = table.shape
    return _sc_lookup(table.reshape(v * d), x, b, s, v, d)
